# Initial kernel scaffold; baseline (speedup 1.0000x reference)
import jax
import jax.numpy as jnp
from jax.experimental import pallas as pl


def kernel(y_pred, y_true):
    raise NotImplementedError("write your pallas kernel here")


# speedup vs baseline: 3.2942x; 3.2942x over previous
import functools

import jax
import jax.numpy as jnp
from jax.experimental import pallas as pl
from jax.experimental.pallas import tpu as pltpu

LANES = 128
TARGET_BLOCK_BYTES = 2 * 1024 * 1024
NUM_CHUNKS = 2
VMEM_LIMIT_BYTES = 48 * 1024 * 1024


def _round_up(x: int, m: int) -> int:
    return (x + m - 1) // m * m


def _fold_rows(x):
    r, w = x.shape
    return jnp.sum(x.reshape(r // 8, 8, w), axis=0)


def _sums_kernel(yp_ref, yt_ref, tp_ref, sp_ref, st_ref, *,
                 tile_r: int, steps_per_chunk: int, rows_total: int,
                 full_blocks: int, any_masked: bool):
    c = pl.program_id(0)
    s = pl.program_id(1)

    @pl.when(s == 0)
    def _init():
        tp_ref[...] = jnp.zeros_like(tp_ref)
        sp_ref[...] = jnp.zeros_like(sp_ref)
        st_ref[...] = jnp.zeros_like(st_ref)

    yp = yp_ref[...].astype(jnp.float32)
    yt = yt_ref[...].astype(jnp.float32)

    def accumulate(ypv, ytv):
        tp_ref[...] += _fold_rows(ytv * ypv)
        sp_ref[...] += _fold_rows(ypv)
        st_ref[...] += _fold_rows(ytv)

    if any_masked:
        gb = c * steps_per_chunk + s

        @pl.when(gb >= full_blocks)
        def _edge():
            row = gb * tile_r + jax.lax.broadcasted_iota(
                jnp.int32, yp.shape, 0)
            valid = row < rows_total
            accumulate(jnp.where(valid, yp, 0.0), jnp.where(valid, yt, 0.0))

        @pl.when(gb < full_blocks)
        def _full():
            accumulate(yp, yt)
    else:
        accumulate(yp, yt)


def _three_sums_pallas(yp2: jax.Array, yt2: jax.Array):
    R, W = yp2.shape
    tile_r = max(8, min(_round_up(R, 8), TARGET_BLOCK_BYTES // (4 * W)))
    tile_r = _round_up(tile_r, 8)

    n_blocks = pl.cdiv(R, tile_r)
    full_blocks = R // tile_r
    chunks = NUM_CHUNKS if n_blocks >= NUM_CHUNKS else 1
    steps = pl.cdiv(n_blocks, chunks)
    any_masked = chunks * steps > full_blocks

    kern = functools.partial(
        _sums_kernel, tile_r=tile_r, steps_per_chunk=steps,
        rows_total=R, full_blocks=full_blocks, any_masked=any_masked)

    def in_map(c, s):
        return (jnp.minimum(c * steps + s, n_blocks - 1), 0)

    out_map = lambda c, s: (c, 0)
    part = jax.ShapeDtypeStruct((chunks * 8, W), jnp.float32)
    in_bytes = yp2.size * yp2.dtype.itemsize + yt2.size * yt2.dtype.itemsize

    tp_p, sp_p, st_p = pl.pallas_call(
        kern,
        out_shape=(part, part, part),
        grid=(chunks, steps),
        in_specs=[pl.BlockSpec((tile_r, W), in_map),
                  pl.BlockSpec((tile_r, W), in_map)],
        out_specs=(pl.BlockSpec((8, W), out_map),
                   pl.BlockSpec((8, W), out_map),
                   pl.BlockSpec((8, W), out_map)),
        compiler_params=pltpu.CompilerParams(
            dimension_semantics=("parallel", "arbitrary"),
            vmem_limit_bytes=VMEM_LIMIT_BYTES),
        cost_estimate=pl.CostEstimate(
            flops=4 * yp2.size, transcendentals=0,
            bytes_accessed=in_bytes + 3 * chunks * 8 * W * 4),
    )(yp2, yt2)

    return jnp.sum(tp_p), jnp.sum(sp_p), jnp.sum(st_p)


def kernel(y_pred: jax.Array, y_true: jax.Array) -> jax.Array:
    beta2 = 1.0
    eps = jnp.float32(1e-6)

    n = y_pred.size
    yp_flat = y_pred.reshape(-1)
    yt_flat = y_true.reshape(-1)

    n_main = (n // LANES) * LANES
    tp = jnp.float32(0.0)
    sp = jnp.float32(0.0)
    st = jnp.float32(0.0)

    if n_main < n:
        ypt = yp_flat[n_main:].astype(jnp.float32)
        ytt = yt_flat[n_main:].astype(jnp.float32)
        tp = tp + jnp.sum(ytt * ypt)
        sp = sp + jnp.sum(ypt)
        st = st + jnp.sum(ytt)

    if n_main > 0:
        yp_main = yp_flat if n_main == n else yp_flat[:n_main]
        yt_main = yt_flat if n_main == n else yt_flat[:n_main]
        W = LANES
        for w in (1024, 512, 256):
            if n_main % w == 0:
                W = w
                break
        R = n_main // W
        tp_k, sp_k, st_k = _three_sums_pallas(
            yp_main.reshape(R, W), yt_main.reshape(R, W))
        tp = tp + tp_k
        sp = sp + sp_k
        st = st + st_k

    fn = sp - tp
    fp = st - tp
    p = tp / (tp + fp + eps)
    r = tp / (tp + fn + eps)
    f1 = (1.0 + beta2) * (p * r) / (beta2 * p + r + eps)
    f1 = jnp.where(jnp.isnan(f1), jnp.zeros_like(f1), f1)
    return (1.0 - f1).astype(jnp.float32)


# speedup vs baseline: 3.5310x; 1.0719x over previous
import functools

import jax
import jax.numpy as jnp
from jax.experimental import pallas as pl
from jax.experimental.pallas import tpu as pltpu

LANES = 128
TARGET_BLOCK_BYTES = 8 * 1024 * 1024
NUM_CHUNKS = 2
VMEM_LIMIT_BYTES = 48 * 1024 * 1024


def _round_up(x: int, m: int) -> int:
    return (x + m - 1) // m * m


def _fold_rows(x):
    r, w = x.shape
    return jnp.sum(x.reshape(r // 8, 8, w), axis=0)


def _sums_kernel(yp_ref, yt_ref, tp_ref, sp_ref, st_ref, *,
                 tile_r: int, steps_per_chunk: int, rows_total: int,
                 full_blocks: int, any_masked: bool):
    c = pl.program_id(0)
    s = pl.program_id(1)

    @pl.when(s == 0)
    def _init():
        tp_ref[...] = jnp.zeros_like(tp_ref)
        sp_ref[...] = jnp.zeros_like(sp_ref)
        st_ref[...] = jnp.zeros_like(st_ref)

    yp = yp_ref[...].astype(jnp.float32)
    yt = yt_ref[...].astype(jnp.float32)

    def accumulate(ypv, ytv):
        tp_ref[...] += _fold_rows(ytv * ypv)
        sp_ref[...] += _fold_rows(ypv)
        st_ref[...] += _fold_rows(ytv)

    if any_masked:
        gb = c * steps_per_chunk + s

        @pl.when(gb >= full_blocks)
        def _edge():
            row = gb * tile_r + jax.lax.broadcasted_iota(
                jnp.int32, yp.shape, 0)
            valid = row < rows_total
            accumulate(jnp.where(valid, yp, 0.0), jnp.where(valid, yt, 0.0))

        @pl.when(gb < full_blocks)
        def _full():
            accumulate(yp, yt)
    else:
        accumulate(yp, yt)


def _three_sums_pallas(yp2: jax.Array, yt2: jax.Array):
    R, W = yp2.shape
    tile_r = max(8, min(_round_up(R, 8), TARGET_BLOCK_BYTES // (4 * W)))
    tile_r = _round_up(tile_r, 8)

    n_blocks = pl.cdiv(R, tile_r)
    full_blocks = R // tile_r
    chunks = NUM_CHUNKS if n_blocks >= NUM_CHUNKS else 1
    steps = pl.cdiv(n_blocks, chunks)
    any_masked = chunks * steps > full_blocks

    kern = functools.partial(
        _sums_kernel, tile_r=tile_r, steps_per_chunk=steps,
        rows_total=R, full_blocks=full_blocks, any_masked=any_masked)

    def in_map(c, s):
        return (jnp.minimum(c * steps + s, n_blocks - 1), 0)

    out_map = lambda c, s: (c, 0)
    part = jax.ShapeDtypeStruct((chunks * 8, W), jnp.float32)
    in_bytes = yp2.size * yp2.dtype.itemsize + yt2.size * yt2.dtype.itemsize

    tp_p, sp_p, st_p = pl.pallas_call(
        kern,
        out_shape=(part, part, part),
        grid=(chunks, steps),
        in_specs=[pl.BlockSpec((tile_r, W), in_map),
                  pl.BlockSpec((tile_r, W), in_map)],
        out_specs=(pl.BlockSpec((8, W), out_map),
                   pl.BlockSpec((8, W), out_map),
                   pl.BlockSpec((8, W), out_map)),
        compiler_params=pltpu.CompilerParams(
            dimension_semantics=("parallel", "arbitrary"),
            vmem_limit_bytes=VMEM_LIMIT_BYTES),
        cost_estimate=pl.CostEstimate(
            flops=4 * yp2.size, transcendentals=0,
            bytes_accessed=in_bytes + 3 * chunks * 8 * W * 4),
    )(yp2, yt2)

    return jnp.sum(tp_p), jnp.sum(sp_p), jnp.sum(st_p)


def kernel(y_pred: jax.Array, y_true: jax.Array) -> jax.Array:
    beta2 = 1.0
    eps = jnp.float32(1e-6)

    n = y_pred.size
    yp_flat = y_pred.reshape(-1)
    yt_flat = y_true.reshape(-1)

    n_main = (n // LANES) * LANES
    tp = jnp.float32(0.0)
    sp = jnp.float32(0.0)
    st = jnp.float32(0.0)

    if n_main < n:
        ypt = yp_flat[n_main:].astype(jnp.float32)
        ytt = yt_flat[n_main:].astype(jnp.float32)
        tp = tp + jnp.sum(ytt * ypt)
        sp = sp + jnp.sum(ypt)
        st = st + jnp.sum(ytt)

    if n_main > 0:
        yp_main = yp_flat if n_main == n else yp_flat[:n_main]
        yt_main = yt_flat if n_main == n else yt_flat[:n_main]
        W = LANES
        for w in (1024, 512, 256):
            if n_main % w == 0:
                W = w
                break
        R = n_main // W
        tp_k, sp_k, st_k = _three_sums_pallas(
            yp_main.reshape(R, W), yt_main.reshape(R, W))
        tp = tp + tp_k
        sp = sp + sp_k
        st = st + st_k

    fn = sp - tp
    fp = st - tp
    p = tp / (tp + fp + eps)
    r = tp / (tp + fn + eps)
    f1 = (1.0 + beta2) * (p * r) / (beta2 * p + r + eps)
    f1 = jnp.where(jnp.isnan(f1), jnp.zeros_like(f1), f1)
    return (1.0 - f1).astype(jnp.float32)


# speedup vs baseline: 3.5747x; 1.0124x over previous
import functools

import jax
import jax.numpy as jnp
from jax.experimental import pallas as pl
from jax.experimental.pallas import tpu as pltpu

LANES = 128
TARGET_BLOCK_BYTES = 4 * 1024 * 1024
NUM_CHUNKS = 2
VMEM_LIMIT_BYTES = 48 * 1024 * 1024


def _round_up(x: int, m: int) -> int:
    return (x + m - 1) // m * m


def _fold_rows(x):
    r, w = x.shape
    return jnp.sum(x.reshape(r // 8, 8, w), axis=0)


def _sums_kernel(yp_ref, yt_ref, tp_ref, sp_ref, st_ref, *,
                 tile_r: int, steps_per_chunk: int, rows_total: int,
                 full_blocks: int, any_masked: bool):
    c = pl.program_id(0)
    s = pl.program_id(1)

    @pl.when(s == 0)
    def _init():
        tp_ref[...] = jnp.zeros_like(tp_ref)
        sp_ref[...] = jnp.zeros_like(sp_ref)
        st_ref[...] = jnp.zeros_like(st_ref)

    yp = yp_ref[...].astype(jnp.float32)
    yt = yt_ref[...].astype(jnp.float32)

    def accumulate(ypv, ytv):
        tp_ref[...] += _fold_rows(ytv * ypv)
        sp_ref[...] += _fold_rows(ypv)
        st_ref[...] += _fold_rows(ytv)

    if any_masked:
        gb = c * steps_per_chunk + s

        @pl.when(gb >= full_blocks)
        def _edge():
            row = gb * tile_r + jax.lax.broadcasted_iota(
                jnp.int32, yp.shape, 0)
            valid = row < rows_total
            accumulate(jnp.where(valid, yp, 0.0), jnp.where(valid, yt, 0.0))

        @pl.when(gb < full_blocks)
        def _full():
            accumulate(yp, yt)
    else:
        accumulate(yp, yt)


def _three_sums_pallas(yp2: jax.Array, yt2: jax.Array):
    R, W = yp2.shape
    tile_r = max(8, min(_round_up(R, 8), TARGET_BLOCK_BYTES // (4 * W)))
    tile_r = _round_up(tile_r, 8)

    n_blocks = pl.cdiv(R, tile_r)
    full_blocks = R // tile_r
    chunks = NUM_CHUNKS if n_blocks >= NUM_CHUNKS else 1
    steps = pl.cdiv(n_blocks, chunks)
    any_masked = chunks * steps > full_blocks

    kern = functools.partial(
        _sums_kernel, tile_r=tile_r, steps_per_chunk=steps,
        rows_total=R, full_blocks=full_blocks, any_masked=any_masked)

    def in_map(c, s):
        return (jnp.minimum(c * steps + s, n_blocks - 1), 0)

    out_map = lambda c, s: (c, 0)
    part = jax.ShapeDtypeStruct((chunks * 8, W), jnp.float32)
    in_bytes = yp2.size * yp2.dtype.itemsize + yt2.size * yt2.dtype.itemsize

    tp_p, sp_p, st_p = pl.pallas_call(
        kern,
        out_shape=(part, part, part),
        grid=(chunks, steps),
        in_specs=[pl.BlockSpec((tile_r, W), in_map),
                  pl.BlockSpec((tile_r, W), in_map)],
        out_specs=(pl.BlockSpec((8, W), out_map),
                   pl.BlockSpec((8, W), out_map),
                   pl.BlockSpec((8, W), out_map)),
        compiler_params=pltpu.CompilerParams(
            dimension_semantics=("parallel", "arbitrary"),
            vmem_limit_bytes=VMEM_LIMIT_BYTES),
        cost_estimate=pl.CostEstimate(
            flops=4 * yp2.size, transcendentals=0,
            bytes_accessed=in_bytes + 3 * chunks * 8 * W * 4),
    )(yp2, yt2)

    return jnp.sum(tp_p), jnp.sum(sp_p), jnp.sum(st_p)


def kernel(y_pred: jax.Array, y_true: jax.Array) -> jax.Array:
    beta2 = 1.0
    eps = jnp.float32(1e-6)

    n = y_pred.size
    yp_flat = y_pred.reshape(-1)
    yt_flat = y_true.reshape(-1)

    n_main = (n // LANES) * LANES
    tp = jnp.float32(0.0)
    sp = jnp.float32(0.0)
    st = jnp.float32(0.0)

    if n_main < n:
        ypt = yp_flat[n_main:].astype(jnp.float32)
        ytt = yt_flat[n_main:].astype(jnp.float32)
        tp = tp + jnp.sum(ytt * ypt)
        sp = sp + jnp.sum(ypt)
        st = st + jnp.sum(ytt)

    if n_main > 0:
        yp_main = yp_flat if n_main == n else yp_flat[:n_main]
        yt_main = yt_flat if n_main == n else yt_flat[:n_main]
        W = LANES
        for w in (1024, 512, 256):
            if n_main % w == 0:
                W = w
                break
        R = n_main // W
        tp_k, sp_k, st_k = _three_sums_pallas(
            yp_main.reshape(R, W), yt_main.reshape(R, W))
        tp = tp + tp_k
        sp = sp + sp_k
        st = st + st_k

    fn = sp - tp
    fp = st - tp
    p = tp / (tp + fp + eps)
    r = tp / (tp + fn + eps)
    f1 = (1.0 + beta2) * (p * r) / (beta2 * p + r + eps)
    f1 = jnp.where(jnp.isnan(f1), jnp.zeros_like(f1), f1)
    return (1.0 - f1).astype(jnp.float32)


# speedup vs baseline: 4.0559x; 1.1346x over previous
import functools

import jax
import jax.numpy as jnp
from jax.experimental import pallas as pl
from jax.experimental.pallas import tpu as pltpu

LANES = 128
TARGET_BLOCK_BYTES = 4 * 1024 * 1024
VMEM_LIMIT_BYTES = 48 * 1024 * 1024


def _round_up(x: int, m: int) -> int:
    return (x + m - 1) // m * m


def _fold_rows(x):
    r, w = x.shape
    return jnp.sum(x.reshape(r // 8, 8, w), axis=0)


def _soft_f1_kernel(tail_ref, yp_ref, yt_ref, out_ref,
                    tp_ref, sp_ref, st_ref, *,
                    tile_r: int, n_steps: int, rows_total: int,
                    full_blocks: int, any_masked: bool,
                    beta2: float, eps: float):
    s = pl.program_id(0)

    @pl.when(s == 0)
    def _init():
        tp_ref[...] = jnp.zeros_like(tp_ref)
        sp_ref[...] = jnp.zeros_like(sp_ref)
        st_ref[...] = jnp.zeros_like(st_ref)

    yp = yp_ref[...].astype(jnp.float32)
    yt = yt_ref[...].astype(jnp.float32)

    def accumulate(ypv, ytv):
        tp_ref[...] += _fold_rows(ytv * ypv)
        sp_ref[...] += _fold_rows(ypv)
        st_ref[...] += _fold_rows(ytv)

    if any_masked:
        @pl.when(s >= full_blocks)
        def _edge():
            row = s * tile_r + jax.lax.broadcasted_iota(
                jnp.int32, yp.shape, 0)
            valid = row < rows_total
            accumulate(jnp.where(valid, yp, 0.0), jnp.where(valid, yt, 0.0))

        @pl.when(s < full_blocks)
        def _full():
            accumulate(yp, yt)
    else:
        accumulate(yp, yt)

    @pl.when(s == n_steps - 1)
    def _finish():
        tp = jnp.sum(tp_ref[...]) + tail_ref[0]
        sp = jnp.sum(sp_ref[...]) + tail_ref[1]
        st = jnp.sum(st_ref[...]) + tail_ref[2]
        epsf = jnp.float32(eps)
        fn = sp - tp
        fp = st - tp
        p = tp / (tp + fp + epsf)
        r = tp / (tp + fn + epsf)
        f1 = (1.0 + beta2) * (p * r) / (beta2 * p + r + epsf)
        f1 = jnp.where(jnp.isnan(f1), jnp.zeros_like(f1), f1)
        out_ref[0, 0] = (1.0 - f1).astype(jnp.float32)


def _soft_f1_pallas(yp2: jax.Array, yt2: jax.Array, tails, beta2, eps):
    R, W = yp2.shape
    tile_r = max(8, min(_round_up(R, 8), TARGET_BLOCK_BYTES // (4 * W)))
    tile_r = _round_up(tile_r, 8)

    n_blocks = pl.cdiv(R, tile_r)
    full_blocks = R // tile_r
    any_masked = n_blocks > full_blocks

    kern = functools.partial(
        _soft_f1_kernel, tile_r=tile_r, n_steps=n_blocks,
        rows_total=R, full_blocks=full_blocks, any_masked=any_masked,
        beta2=beta2, eps=eps)

    in_map = lambda s: (s, 0)
    in_bytes = yp2.size * yp2.dtype.itemsize + yt2.size * yt2.dtype.itemsize

    out = pl.pallas_call(
        kern,
        out_shape=jax.ShapeDtypeStruct((1, 1), jnp.float32),
        grid=(n_blocks,),
        in_specs=[pl.BlockSpec(memory_space=pltpu.SMEM),
                  pl.BlockSpec((tile_r, W), in_map),
                  pl.BlockSpec((tile_r, W), in_map)],
        out_specs=pl.BlockSpec(memory_space=pltpu.SMEM),
        scratch_shapes=[pltpu.VMEM((8, W), jnp.float32)] * 3,
        compiler_params=pltpu.CompilerParams(
            dimension_semantics=("arbitrary",),
            vmem_limit_bytes=VMEM_LIMIT_BYTES),
        cost_estimate=pl.CostEstimate(
            flops=4 * yp2.size, transcendentals=0,
            bytes_accessed=in_bytes + 4),
    )(tails, yp2, yt2)

    return out[0, 0]


def kernel(y_pred: jax.Array, y_true: jax.Array) -> jax.Array:
    beta2 = 1.0
    eps = 1e-6

    n = y_pred.size
    yp_flat = y_pred.reshape(-1)
    yt_flat = y_true.reshape(-1)

    n_main = (n // LANES) * LANES

    if n_main < n:
        ypt = yp_flat[n_main:].astype(jnp.float32)
        ytt = yt_flat[n_main:].astype(jnp.float32)
        tails = jnp.stack(
            [jnp.sum(ytt * ypt), jnp.sum(ypt), jnp.sum(ytt)])
    else:
        tails = jnp.zeros((3,), jnp.float32)

    if n_main == 0:
        tp, sp, st = tails[0], tails[1], tails[2]
        epsf = jnp.float32(eps)
        fn = sp - tp
        fp = st - tp
        p = tp / (tp + fp + epsf)
        r = tp / (tp + fn + epsf)
        f1 = (1.0 + beta2) * (p * r) / (beta2 * p + r + epsf)
        f1 = jnp.where(jnp.isnan(f1), jnp.zeros_like(f1), f1)
        return (1.0 - f1).astype(jnp.float32)

    yp_main = yp_flat if n_main == n else yp_flat[:n_main]
    yt_main = yt_flat if n_main == n else yt_flat[:n_main]
    W = LANES
    for w in (1024, 512, 256):
        if n_main % w == 0:
            W = w
            break
    R = n_main // W
    return _soft_f1_pallas(
        yp_main.reshape(R, W), yt_main.reshape(R, W), tails, beta2, eps)
